# baseline (speedup 1.0000x reference)
import functools

import jax
import jax.numpy as jnp
from jax import lax
from jax.experimental import pallas as pl
from jax.experimental.pallas import tpu as pltpu
from jax.experimental.pallas import tpu_sc as plsc

TI = 128
SC_H = 8
NW = 32
ROWS_W = 2048 // NW


def _bf16_round(v):
    u = lax.bitcast_convert_type(v, jnp.uint32)
    r = (u + jnp.uint32(0x7FFF) + ((u >> 16) & jnp.uint32(1))) & jnp.uint32(0xFFFF0000)
    return lax.bitcast_convert_type(r, jnp.float32)


def _layernorm(h, g, b):
    m = h.mean(-1, keepdims=True)
    v = h.var(-1, keepdims=True)
    return (h - m) / jnp.sqrt(v + 1e-5) * g + b


def _mlp(x, Ws, bs_, gs, betas):
    h = x
    for i in range(3):
        h = _layernorm(h, gs[i], betas[i])
        h = jnp.dot(h, Ws[i], preferred_element_type=jnp.float32) + bs_[i]
        if i < 2:
            h = 0.5 * h * (1.0 + lax.erf(h * 0.7071067811865476))
    return h


def _sc_partial(hm_hbm, wh_hbm, out_hbm, acc_v, buf_v, wh_v, sem0, sem1):
    wid = lax.axis_index("s") * 2 + lax.axis_index("c")
    b = wid // 16
    e0 = (wid % 16) * (ROWS_W * 588)
    h0 = 16 - SC_H
    NB = ROWS_W * 588 // 16
    pltpu.sync_copy(wh_hbm, wh_v)
    wvec = wh_v[...]

    def zero8(j, carry):
        for u in range(8):
            acc_v[pl.ds((j * 8 + u) * 16, 16)] = jnp.zeros((16,), jnp.float32)
        return carry
    lax.fori_loop(0, NB // 8, zero8, 0)

    sems = (sem0, sem1)
    copies = [None, None]
    copies[0] = pltpu.async_copy(
        hm_hbm.at[3, b, h0, pl.ds(e0, ROWS_W * 588)], buf_v.at[0], sems[0])
    for p in range(SC_H):
        slot = p % 2
        if p + 1 < SC_H:
            nslot = (p + 1) % 2
            copies[nslot] = pltpu.async_copy(
                hm_hbm.at[3, b, h0 + p + 1, pl.ds(e0, ROWS_W * 588)],
                buf_v.at[nslot], sems[nslot])
        copies[slot].wait()
        w = wvec[p]

        def fma8(j, carry, slot=slot, w=w):
            for u in range(8):
                o = (j * 8 + u) * 16
                v = _bf16_round(buf_v[slot, pl.ds(o, 16)])
                acc_v[pl.ds(o, 16)] = acc_v[pl.ds(o, 16)] + w * v
            return carry
        lax.fori_loop(0, NB // 8, fma8, 0)

    pltpu.sync_copy(acc_v, out_hbm.at[pl.ds(wid * ROWS_W * 588, ROWS_W * 588)])


def _sc_partial_x(heatmap, wh3):
    hm_flat = heatmap.reshape(4, 2, 16, 1024 * 588)
    mesh = plsc.VectorSubcoreMesh(core_axis_name="c", subcore_axis_name="s",
                                  num_cores=2, num_subcores=16)
    out = pl.kernel(
        _sc_partial,
        mesh=mesh,
        out_type=jax.ShapeDtypeStruct((2048 * 588,), jnp.float32),
        scratch_types=[
            pltpu.VMEM((ROWS_W * 588,), jnp.float32),
            pltpu.VMEM((2, ROWS_W * 588), jnp.float32),
            pltpu.VMEM((16,), jnp.float32),
            pltpu.SemaphoreType.DMA,
            pltpu.SemaphoreType.DMA,
        ],
    )(hm_flat, wh3)
    return out.reshape(2048, 588)


def _tc_partial_body(hm012_ref, hm3_ref, wh_ref, out_ref):
    wv = _bf16_round(wh_ref[...]).reshape(4, 16)
    acc = jnp.zeros((TI, 588), jnp.float32)
    for l in range(3):
        for h in range(16):
            p = hm012_ref[l, 0, h].astype(jnp.bfloat16).astype(jnp.float32)
            acc = acc + p * wv[l, h]
    for h in range(16 - SC_H):
        p = hm3_ref[0, 0, h].astype(jnp.bfloat16).astype(jnp.float32)
        acc = acc + p * wv[3, h]
    out_ref[...] = acc


def _combine_body(tcx_ref, scx_ref, amask_ref, wk_ref, w10_ref, b10_ref,
                  w11_ref, b11_ref, w12_ref, b12_ref, w20_ref, b20_ref,
                  w21_ref, b21_ref, w22_ref, b22_ref, g0_ref, be0_ref,
                  g1_ref, be1_ref, g2_ref, be2_ref, bh_bk_ref, out_ref):
    x = tcx_ref[...] + scx_ref[...] + bh_bk_ref[0, 0]

    xb = _bf16_round(x)
    wkb = _bf16_round(wk_ref[...]).reshape(1, 588)
    logits = jnp.sum(xb * wkb, axis=1, keepdims=True)
    logits = logits + bh_bk_ref[0, 1]
    mask = logits > 0.0

    gs = (g0_ref[...], g1_ref[...], g2_ref[...])
    betas = (be0_ref[...], be1_ref[...], be2_ref[...])
    p1 = _mlp(x, (w10_ref[...], w11_ref[...], w12_ref[...]),
              (b10_ref[...], b11_ref[...], b12_ref[...]), gs, betas)
    p2 = _mlp(x, (w20_ref[...], w21_ref[...], w22_ref[...]),
              (b20_ref[...], b21_ref[...], b22_ref[...]), gs, betas)
    out = jnp.where(mask, p1, p2)
    out = jax.nn.sigmoid(out) * amask_ref[...]

    x1 = out[:, 0:1]
    y1 = out[:, 1:2]
    x2 = x1 + out[:, 2:3]
    y2 = y1 + out[:, 3:4]
    zeros = jnp.zeros((TI, 3), jnp.float32)
    out_ref[...] = jnp.concatenate([x1, y1, x2, y2, logits, zeros], axis=1)


def kernel(heatmap, attention_valid_mask, Wh, bh, Wk, bk,
           W1_0, b1_0, W1_1, b1_1, W1_2, b1_2,
           W2_0, b2_0, W2_1, b2_1, W2_2, b2_2,
           g_0, beta_0, g_1, beta_1, g_2, beta_2):
    num_layer, bs, num_heads, input_len, encoder_len = heatmap.shape
    nt = input_len // TI
    rows = bs * input_len

    amask = attention_valid_mask.reshape(rows, 1)
    whr = Wh.reshape(1, 64)
    wh3 = jnp.pad(_bf16_round(Wh[64 - SC_H:, 0]), (0, 16 - SC_H))
    bh_bk = jnp.stack([bh[0], bk[0]]).reshape(1, 2)

    sc_x = _sc_partial_x(heatmap, wh3)

    tc_x = pl.pallas_call(
        _tc_partial_body,
        grid=(bs, nt),
        in_specs=[
            pl.BlockSpec((3, 1, num_heads, TI, encoder_len),
                         lambda b, t: (0, b, 0, t, 0)),
            pl.BlockSpec((1, 1, 16 - SC_H, TI, encoder_len),
                         lambda b, t: (3, b, 0, t, 0)),
            pl.BlockSpec((1, 64), lambda b, t: (0, 0)),
        ],
        out_specs=pl.BlockSpec((TI, encoder_len), lambda b, t: (b * nt + t, 0)),
        out_shape=jax.ShapeDtypeStruct((rows, encoder_len), jnp.float32),
    )(heatmap, heatmap, whr)

    def rep(_b, _t):
        return (0, 0)

    out_all = pl.pallas_call(
        _combine_body,
        grid=(bs, nt),
        in_specs=[
            pl.BlockSpec((TI, encoder_len), lambda b, t: (b * nt + t, 0)),
            pl.BlockSpec((TI, encoder_len), lambda b, t: (b * nt + t, 0)),
            pl.BlockSpec((TI, 1), lambda b, t: (b * nt + t, 0)),
            pl.BlockSpec((encoder_len, 1), rep),
            pl.BlockSpec(W1_0.shape, rep), pl.BlockSpec((1, 256), rep),
            pl.BlockSpec(W1_1.shape, rep), pl.BlockSpec((1, 256), rep),
            pl.BlockSpec(W1_2.shape, rep), pl.BlockSpec((1, 4), rep),
            pl.BlockSpec(W2_0.shape, rep), pl.BlockSpec((1, 256), rep),
            pl.BlockSpec(W2_1.shape, rep), pl.BlockSpec((1, 256), rep),
            pl.BlockSpec(W2_2.shape, rep), pl.BlockSpec((1, 4), rep),
            pl.BlockSpec((1, 588), rep), pl.BlockSpec((1, 588), rep),
            pl.BlockSpec((1, 256), rep), pl.BlockSpec((1, 256), rep),
            pl.BlockSpec((1, 256), rep), pl.BlockSpec((1, 256), rep),
            pl.BlockSpec((1, 2), rep),
        ],
        out_specs=pl.BlockSpec((TI, 8), lambda b, t: (b * nt + t, 0)),
        out_shape=jax.ShapeDtypeStruct((rows, 8), jnp.float32),
    )(tc_x, sc_x, amask, Wk,
      W1_0, b1_0.reshape(1, 256), W1_1, b1_1.reshape(1, 256),
      W1_2, b1_2.reshape(1, 4),
      W2_0, b2_0.reshape(1, 256), W2_1, b2_1.reshape(1, 256),
      W2_2, b2_2.reshape(1, 4),
      g_0.reshape(1, 588), beta_0.reshape(1, 588),
      g_1.reshape(1, 256), beta_1.reshape(1, 256),
      g_2.reshape(1, 256), beta_2.reshape(1, 256),
      bh_bk)

    o = out_all.reshape(bs, input_len, 8)
    return (o[:, :, 0], o[:, :, 1], o[:, :, 2], o[:, :, 3],
            out_all[:, 4])


# speedup vs baseline: 1.0199x; 1.0199x over previous
import functools

import jax
import jax.numpy as jnp
from jax import lax
from jax.experimental import pallas as pl
from jax.experimental.pallas import tpu as pltpu
from jax.experimental.pallas import tpu_sc as plsc

TI = 128
SC_H = 8
NW = 32
ROWS_W = 2048 // NW


def _bf16_round(v):
    u = lax.bitcast_convert_type(v, jnp.uint32)
    r = (u + jnp.uint32(0x7FFF) + ((u >> 16) & jnp.uint32(1))) & jnp.uint32(0xFFFF0000)
    return lax.bitcast_convert_type(r, jnp.float32)


def _layernorm(h, g, b):
    m = h.mean(-1, keepdims=True)
    v = h.var(-1, keepdims=True)
    return (h - m) / jnp.sqrt(v + 1e-5) * g + b


def _mlp(x, Ws, bs_, gs, betas):
    h = x
    for i in range(3):
        h = _layernorm(h, gs[i], betas[i])
        h = jnp.dot(h.astype(jnp.bfloat16), Ws[i].astype(jnp.bfloat16),
                    preferred_element_type=jnp.float32) + bs_[i]
        if i < 2:
            h = 0.5 * h * (1.0 + lax.erf(h * 0.7071067811865476))
    return h


def _sc_partial(hm_hbm, wh_hbm, out_hbm, acc_v, buf_v, wh_v, sem0, sem1):
    wid = lax.axis_index("s") * 2 + lax.axis_index("c")
    b = wid // 16
    e0 = (wid % 16) * (ROWS_W * 588)
    h0 = 16 - SC_H
    NB = ROWS_W * 588 // 16
    pltpu.sync_copy(wh_hbm, wh_v)
    wvec = wh_v[...]

    def zero8(j, carry):
        for u in range(8):
            acc_v[pl.ds((j * 8 + u) * 16, 16)] = jnp.zeros((16,), jnp.float32)
        return carry
    lax.fori_loop(0, NB // 8, zero8, 0)

    sems = (sem0, sem1)
    copies = [None, None]
    copies[0] = pltpu.async_copy(
        hm_hbm.at[3, b, h0, pl.ds(e0, ROWS_W * 588)], buf_v.at[0], sems[0])
    for p in range(SC_H):
        slot = p % 2
        if p + 1 < SC_H:
            nslot = (p + 1) % 2
            copies[nslot] = pltpu.async_copy(
                hm_hbm.at[3, b, h0 + p + 1, pl.ds(e0, ROWS_W * 588)],
                buf_v.at[nslot], sems[nslot])
        copies[slot].wait()
        w = wvec[p]

        def fma8(j, carry, slot=slot, w=w):
            for u in range(8):
                o = (j * 8 + u) * 16
                v = _bf16_round(buf_v[slot, pl.ds(o, 16)])
                acc_v[pl.ds(o, 16)] = acc_v[pl.ds(o, 16)] + w * v
            return carry
        lax.fori_loop(0, NB // 8, fma8, 0)

    pltpu.sync_copy(acc_v, out_hbm.at[pl.ds(wid * ROWS_W * 588, ROWS_W * 588)])


def _sc_partial_x(heatmap, wh3):
    hm_flat = heatmap.reshape(4, 2, 16, 1024 * 588)
    mesh = plsc.VectorSubcoreMesh(core_axis_name="c", subcore_axis_name="s",
                                  num_cores=2, num_subcores=16)
    out = pl.kernel(
        _sc_partial,
        mesh=mesh,
        out_type=jax.ShapeDtypeStruct((2048 * 588,), jnp.float32),
        scratch_types=[
            pltpu.VMEM((ROWS_W * 588,), jnp.float32),
            pltpu.VMEM((2, ROWS_W * 588), jnp.float32),
            pltpu.VMEM((16,), jnp.float32),
            pltpu.SemaphoreType.DMA,
            pltpu.SemaphoreType.DMA,
        ],
    )(hm_flat, wh3)
    return out.reshape(2048, 588)


def _tc_partial_body(whb_ref, hm_ref, out_ref):
    k = pl.program_id(1)
    w = whb_ref[k]
    p = hm_ref[0, 0, 0].astype(jnp.bfloat16).astype(jnp.float32)
    contrib = p * w

    @pl.when(k == 0)
    def _():
        out_ref[...] = contrib

    @pl.when(k > 0)
    def _():
        out_ref[...] = out_ref[...] + contrib


def _combine_body(tcx_ref, scx_ref, amask_ref, wk_ref, w10_ref, b10_ref,
                  w11_ref, b11_ref, w12_ref, b12_ref, w20_ref, b20_ref,
                  w21_ref, b21_ref, w22_ref, b22_ref, g0_ref, be0_ref,
                  g1_ref, be1_ref, g2_ref, be2_ref, bh_bk_ref, out_ref):
    x = tcx_ref[...] + scx_ref[...] + bh_bk_ref[0, 0]

    xb = _bf16_round(x)
    wkb = _bf16_round(wk_ref[...]).reshape(1, 588)
    logits = jnp.sum(xb * wkb, axis=1, keepdims=True)
    logits = logits + bh_bk_ref[0, 1]
    mask = logits > 0.0

    gs = (g0_ref[...], g1_ref[...], g2_ref[...])
    betas = (be0_ref[...], be1_ref[...], be2_ref[...])
    p1 = _mlp(x, (w10_ref[...], w11_ref[...], w12_ref[...]),
              (b10_ref[...], b11_ref[...], b12_ref[...]), gs, betas)
    p2 = _mlp(x, (w20_ref[...], w21_ref[...], w22_ref[...]),
              (b20_ref[...], b21_ref[...], b22_ref[...]), gs, betas)
    out = jnp.where(mask, p1, p2)
    out = jax.nn.sigmoid(out) * amask_ref[...]

    x1 = out[:, 0:1]
    y1 = out[:, 1:2]
    x2 = x1 + out[:, 2:3]
    y2 = y1 + out[:, 3:4]
    zeros = jnp.zeros((2048, 3), jnp.float32)
    out_ref[...] = jnp.concatenate([x1, y1, x2, y2, logits, zeros], axis=1)


def kernel(heatmap, attention_valid_mask, Wh, bh, Wk, bk,
           W1_0, b1_0, W1_1, b1_1, W1_2, b1_2,
           W2_0, b2_0, W2_1, b2_1, W2_2, b2_2,
           g_0, beta_0, g_1, beta_1, g_2, beta_2):
    num_layer, bs, num_heads, input_len, encoder_len = heatmap.shape
    nt = input_len // TI
    rows = bs * input_len

    amask = attention_valid_mask.reshape(rows, 1)
    whr = Wh.reshape(1, 64)
    wh3 = jnp.pad(_bf16_round(Wh[64 - SC_H:, 0]), (0, 16 - SC_H))
    bh_bk = jnp.stack([bh[0], bk[0]]).reshape(1, 2)

    sc_x = _sc_partial_x(heatmap, wh3)

    NK = 64 - SC_H
    whb = _bf16_round(Wh[:, 0])

    tc_x = pl.pallas_call(
        _tc_partial_body,
        grid=(bs, NK),
        in_specs=[
            pl.BlockSpec(memory_space=pltpu.MemorySpace.SMEM),
            pl.BlockSpec((1, 1, 1, input_len, encoder_len),
                         lambda b, k: (k // 16, b, k % 16, 0, 0)),
        ],
        out_specs=pl.BlockSpec((input_len, encoder_len), lambda b, k: (b, 0)),
        out_shape=jax.ShapeDtypeStruct((rows, encoder_len), jnp.float32),
    )(whb, heatmap)

    def rep(_):
        return (0, 0)

    out_all = pl.pallas_call(
        _combine_body,
        grid=(1,),
        in_specs=[
            pl.BlockSpec((rows, encoder_len), rep),
            pl.BlockSpec((rows, encoder_len), rep),
            pl.BlockSpec((rows, 1), rep),
            pl.BlockSpec((encoder_len, 1), rep),
            pl.BlockSpec(W1_0.shape, rep), pl.BlockSpec((1, 256), rep),
            pl.BlockSpec(W1_1.shape, rep), pl.BlockSpec((1, 256), rep),
            pl.BlockSpec(W1_2.shape, rep), pl.BlockSpec((1, 4), rep),
            pl.BlockSpec(W2_0.shape, rep), pl.BlockSpec((1, 256), rep),
            pl.BlockSpec(W2_1.shape, rep), pl.BlockSpec((1, 256), rep),
            pl.BlockSpec(W2_2.shape, rep), pl.BlockSpec((1, 4), rep),
            pl.BlockSpec((1, 588), rep), pl.BlockSpec((1, 588), rep),
            pl.BlockSpec((1, 256), rep), pl.BlockSpec((1, 256), rep),
            pl.BlockSpec((1, 256), rep), pl.BlockSpec((1, 256), rep),
            pl.BlockSpec((1, 2), rep),
        ],
        out_specs=pl.BlockSpec((rows, 8), rep),
        out_shape=jax.ShapeDtypeStruct((rows, 8), jnp.float32),
    )(tc_x, sc_x, amask, Wk,
      W1_0, b1_0.reshape(1, 256), W1_1, b1_1.reshape(1, 256),
      W1_2, b1_2.reshape(1, 4),
      W2_0, b2_0.reshape(1, 256), W2_1, b2_1.reshape(1, 256),
      W2_2, b2_2.reshape(1, 4),
      g_0.reshape(1, 588), beta_0.reshape(1, 588),
      g_1.reshape(1, 256), beta_1.reshape(1, 256),
      g_2.reshape(1, 256), beta_2.reshape(1, 256),
      bh_bk)

    o = out_all.reshape(bs, input_len, 8)
    return (o[:, :, 0], o[:, :, 1], o[:, :, 2], o[:, :, 3],
            out_all[:, 4])


# speedup vs baseline: 2.2921x; 2.2473x over previous
import jax
import jax.numpy as jnp
from jax import lax
from jax.experimental import pallas as pl
from jax.experimental.pallas import tpu as pltpu

IL = 1024


def _bf16_round(v):
    u = lax.bitcast_convert_type(v, jnp.uint32)
    r = (u + jnp.uint32(0x7FFF) + ((u >> 16) & jnp.uint32(1))) & jnp.uint32(0xFFFF0000)
    return lax.bitcast_convert_type(r, jnp.float32)


def _layernorm(h, g, b):
    m = h.mean(-1, keepdims=True)
    v = h.var(-1, keepdims=True)
    return (h - m) / jnp.sqrt(v + 1e-5) * g + b


def _mlp(x, Ws, bs_, gs, betas):
    h = x
    for i in range(3):
        h = _layernorm(h, gs[i], betas[i])
        h = jnp.dot(h.astype(jnp.bfloat16), Ws[i].astype(jnp.bfloat16),
                    preferred_element_type=jnp.float32) + bs_[i]
        if i < 2:
            h = 0.5 * h * (1.0 + lax.erf(h * 0.7071067811865476))
    return h


def _body(whb_ref, hm_ref, amask_ref, wk_ref, w10_ref, b10_ref, w11_ref,
          b11_ref, w12_ref, b12_ref, w20_ref, b20_ref, w21_ref, b21_ref,
          w22_ref, b22_ref, g0_ref, be0_ref, g1_ref, be1_ref, g2_ref,
          be2_ref, bh_bk_ref, out_ref, acc_ref):
    step = pl.program_id(1)
    k0 = step * 8

    contrib = jnp.zeros((IL, 588), jnp.float32)
    for h in range(8):
        p = hm_ref[0, 0, h].astype(jnp.bfloat16).astype(jnp.float32)
        contrib = contrib + p * whb_ref[k0 + h]

    @pl.when(step == 0)
    def _():
        acc_ref[...] = contrib

    @pl.when(step > 0)
    def _():
        acc_ref[...] = acc_ref[...] + contrib

    @pl.when(step == 7)
    def _():
        x = acc_ref[...] + bh_bk_ref[0, 0]

        xb = _bf16_round(x)
        wkb = _bf16_round(wk_ref[...]).reshape(1, 588)
        logits = jnp.sum(xb * wkb, axis=1, keepdims=True)
        logits = logits + bh_bk_ref[0, 1]
        mask = logits > 0.0

        gs = (g0_ref[...], g1_ref[...], g2_ref[...])
        betas = (be0_ref[...], be1_ref[...], be2_ref[...])
        p1 = _mlp(x, (w10_ref[...], w11_ref[...], w12_ref[...]),
                  (b10_ref[...], b11_ref[...], b12_ref[...]), gs, betas)
        p2 = _mlp(x, (w20_ref[...], w21_ref[...], w22_ref[...]),
                  (b20_ref[...], b21_ref[...], b22_ref[...]), gs, betas)
        out = jnp.where(mask, p1, p2)
        out = jax.nn.sigmoid(out) * amask_ref[...]

        x1 = out[:, 0:1]
        y1 = out[:, 1:2]
        x2 = x1 + out[:, 2:3]
        y2 = y1 + out[:, 3:4]
        zeros = jnp.zeros((IL, 3), jnp.float32)
        out_ref[...] = jnp.concatenate([x1, y1, x2, y2, logits, zeros],
                                       axis=1)


def kernel(heatmap, attention_valid_mask, Wh, bh, Wk, bk,
           W1_0, b1_0, W1_1, b1_1, W1_2, b1_2,
           W2_0, b2_0, W2_1, b2_1, W2_2, b2_2,
           g_0, beta_0, g_1, beta_1, g_2, beta_2):
    num_layer, bs, num_heads, input_len, encoder_len = heatmap.shape
    rows = bs * input_len

    amask = attention_valid_mask.reshape(rows, 1)
    whb = _bf16_round(Wh[:, 0])
    bh_bk = jnp.stack([bh[0], bk[0]]).reshape(1, 2)

    def rep(_b, _l):
        return (0, 0)

    out_all = pl.pallas_call(
        _body,
        grid=(bs, 2 * num_layer),
        in_specs=[
            pl.BlockSpec(memory_space=pltpu.MemorySpace.SMEM),
            pl.BlockSpec((1, 1, num_heads // 2, input_len, encoder_len),
                         lambda b, s: (s // 2, b, s % 2, 0, 0)),
            pl.BlockSpec((input_len, 1), lambda b, l: (b, 0)),
            pl.BlockSpec((encoder_len, 1), rep),
            pl.BlockSpec(W1_0.shape, rep), pl.BlockSpec((1, 256), rep),
            pl.BlockSpec(W1_1.shape, rep), pl.BlockSpec((1, 256), rep),
            pl.BlockSpec(W1_2.shape, rep), pl.BlockSpec((1, 4), rep),
            pl.BlockSpec(W2_0.shape, rep), pl.BlockSpec((1, 256), rep),
            pl.BlockSpec(W2_1.shape, rep), pl.BlockSpec((1, 256), rep),
            pl.BlockSpec(W2_2.shape, rep), pl.BlockSpec((1, 4), rep),
            pl.BlockSpec((1, 588), rep), pl.BlockSpec((1, 588), rep),
            pl.BlockSpec((1, 256), rep), pl.BlockSpec((1, 256), rep),
            pl.BlockSpec((1, 256), rep), pl.BlockSpec((1, 256), rep),
            pl.BlockSpec((1, 2), rep),
        ],
        out_specs=pl.BlockSpec((input_len, 8), lambda b, l: (b, 0)),
        out_shape=jax.ShapeDtypeStruct((rows, 8), jnp.float32),
        scratch_shapes=[pltpu.VMEM((input_len, encoder_len), jnp.float32)],
    )(whb, heatmap, amask, Wk,
      W1_0, b1_0.reshape(1, 256), W1_1, b1_1.reshape(1, 256),
      W1_2, b1_2.reshape(1, 4),
      W2_0, b2_0.reshape(1, 256), W2_1, b2_1.reshape(1, 256),
      W2_2, b2_2.reshape(1, 4),
      g_0.reshape(1, 588), beta_0.reshape(1, 588),
      g_1.reshape(1, 256), beta_1.reshape(1, 256),
      g_2.reshape(1, 256), beta_2.reshape(1, 256),
      bh_bk)

    o = out_all.reshape(bs, input_len, 8)
    return (o[:, :, 0], o[:, :, 1], o[:, :, 2], o[:, :, 3],
            out_all[:, 4])
